# baseline (speedup 1.0000x reference)
import jax
import jax.numpy as jnp
from jax import lax
from jax.experimental import pallas as pl
from jax.experimental.pallas import tpu as pltpu
from jax.experimental.pallas import tpu_sc as plsc

_VOCAB = 1000000
_EMBED = 64
_BATCH = 16384
_SEQ = 200
_RPB = 64
_NUM_BLOCKS = _BATCH // _RPB
_NC, _NS = 2, 16
_NW = _NC * _NS
_BPW = _NUM_BLOCKS // _NW
_HALF = _BATCH // 2
_HBLK = _NUM_BLOCKS // 2


def _sc_body(x_hbm, tp_hbm, out_hbm, xrow_v, idxe_v, idxo_v, cnte_v,
             acce_v, acco_v, t1_v, sem_g):
    wid = lax.axis_index("s") * _NC + lax.axis_index("c")
    lanes = lax.iota(jnp.int32, 16)
    zi = jnp.zeros((16,), jnp.int32)
    zf = jnp.zeros((16,), jnp.float32)

    pltpu.sync_copy(tp_hbm.at[pl.ds(0, 1)], t1_v)

    def _stage_build(g, slot):
        pltpu.sync_copy(x_hbm.at[pl.ds(g * _RPB, _RPB)], xrow_v.at[slot])
        for k in range(5):
            cnte_v[slot, pl.ds(16 * k, 16)] = zi

        def _build(s, c):
            cols = zi + s
            for k in range(4):
                ln = lanes + (16 * k)
                v = plsc.load_gather(xrow_v.at[slot], [ln, cols])
                u = lax.shift_right_logical(v, 1)
                odd = (v & 1) == 1
                idxe_v[slot, s, pl.ds(16 * k, 16)] = jnp.where(odd, 0, u)
                idxo_v[slot, s, pl.ds(16 * k, 16)] = jnp.where(odd, u, 0)
                ce = cnte_v[slot, pl.ds(16 * k, 16)]
                cnte_v[slot, pl.ds(16 * k, 16)] = (
                    ce + jnp.logical_not(odd).astype(jnp.int32))
            return c
        lax.fori_loop(0, _SEQ, _build, 0)

    _stage_build(wid * _BPW, 0)

    for t in range(_BPW):
        slot = t % 2
        g = wid * _BPW + t

        def _zacc(i, c):
            for m in range(8):
                acce_v[i, pl.ds(16 * m, 16)] = zf
                acco_v[i, pl.ds(16 * m, 16)] = zf
            return c
        lax.fori_loop(0, _RPB, _zacc, 0)

        def _fire(k, c):
            pltpu.async_copy(
                tp_hbm.at[idxe_v.at[slot, k]], acce_v, sem_g, add=True)
            pltpu.async_copy(
                tp_hbm.at[idxo_v.at[slot, k]], acco_v, sem_g, add=True)
            return c
        lax.fori_loop(0, _SEQ, _fire, 0)

        if t + 1 < _BPW:
            _stage_build(g + 1, 1 - slot)

        def _drain(k, c):
            pltpu.make_async_copy(
                tp_hbm.at[idxe_v.at[slot, 0]], acce_v, sem_g).wait()
            pltpu.make_async_copy(
                tp_hbm.at[idxe_v.at[slot, 0]], acco_v, sem_g).wait()
            return c
        lax.fori_loop(0, _SEQ, _drain, 0)

        def _comb(i, c):
            c16 = cnte_v[slot, pl.ds(i, 16)]
            d_o = c16[0].astype(jnp.float32)
            for m in range(4):
                acce_v[i, pl.ds(16 * m, 16)] = (
                    acce_v[i, pl.ds(16 * m, 16)]
                    + acco_v[i, pl.ds(64 + 16 * m, 16)]
                    - d_o * t1_v[0, pl.ds(64 + 16 * m, 16)])
            return c
        lax.fori_loop(0, _RPB, _comb, 0)

        pltpu.sync_copy(
            acce_v.at[:, pl.ds(0, _EMBED)],
            out_hbm.at[pl.ds((g % _HBLK) * _RPB, _RPB),
                       pl.ds(_EMBED * (g // _HBLK), _EMBED)])


def _sc_pool(x, table_pairs):
    mesh = plsc.VectorSubcoreMesh(core_axis_name="c", subcore_axis_name="s")
    f = pl.kernel(
        _sc_body,
        out_type=jax.ShapeDtypeStruct((_HALF, 2 * _EMBED), jnp.float32),
        mesh=mesh,
        scratch_types=[
            pltpu.VMEM((2, _RPB, _SEQ), jnp.int32),
            pltpu.VMEM((2, _SEQ, _RPB), jnp.int32),
            pltpu.VMEM((2, _SEQ, _RPB), jnp.int32),
            pltpu.VMEM((2, _RPB + 16), jnp.int32),
            pltpu.VMEM((_RPB, 2 * _EMBED), jnp.float32),
            pltpu.VMEM((_RPB, 2 * _EMBED), jnp.float32),
            pltpu.VMEM((1, 2 * _EMBED), jnp.float32),
            pltpu.SemaphoreType.DMA,
        ],
        compiler_params=pltpu.CompilerParams(
            use_tc_tiling_on_sc=False, needs_layout_passes=False),
    )
    return f(x, table_pairs)


def _tc_head_body(xa_ref, xb_ref, sp_ref, w1p_ref, b1p_ref, w2p_ref, b2_ref,
                  o_ref):
    cnt_a = jnp.sum((xa_ref[...] != 0).astype(jnp.float32), axis=1,
                    keepdims=True)
    cnt_b = jnp.sum((xb_ref[...] != 0).astype(jnp.float32), axis=1,
                    keepdims=True)
    n = sp_ref.shape[0]
    inv = jnp.concatenate(
        [jnp.broadcast_to(1.0 / jnp.maximum(cnt_a, 1.0), (n, _EMBED)),
         jnp.broadcast_to(1.0 / jnp.maximum(cnt_b, 1.0), (n, _EMBED))],
        axis=1)
    pooled = sp_ref[...] * inv
    h = jnp.dot(pooled, w1p_ref[...], preferred_element_type=jnp.float32)
    h = jnp.maximum(h + b1p_ref[...], 0.0)
    o_ref[...] = (
        jnp.dot(h, w2p_ref[...], preferred_element_type=jnp.float32)
        + b2_ref[...])


def _tc_head(x, sp, w1p, b1p, w2p, b2):
    blk = 1024
    nblk = _HALF // blk
    return pl.pallas_call(
        _tc_head_body,
        grid=(nblk,),
        in_specs=[
            pl.BlockSpec((blk, _SEQ), lambda i: (i, 0)),
            pl.BlockSpec((blk, _SEQ), lambda i: (i + nblk, 0)),
            pl.BlockSpec((blk, 2 * _EMBED), lambda i: (i, 0)),
            pl.BlockSpec((2 * _EMBED, 2 * _EMBED), lambda i: (0, 0)),
            pl.BlockSpec((1, 2 * _EMBED), lambda i: (0, 0)),
            pl.BlockSpec((2 * _EMBED, 2), lambda i: (0, 0)),
            pl.BlockSpec((1, 2), lambda i: (0, 0)),
        ],
        out_specs=pl.BlockSpec((blk, 2), lambda i: (i, 0)),
        out_shape=jax.ShapeDtypeStruct((_HALF, 2), jnp.float32),
    )(x, x, sp, w1p, b1p, w2p, b2)


def kernel(x, table, W1, b1, W2, b2):
    table_pairs = table.reshape(_VOCAB // 2, 2 * _EMBED)
    sp = _sc_pool(x, table_pairs)
    z = jnp.zeros((_EMBED, _EMBED), jnp.float32)
    w1p = jnp.block([[W1.T, z], [z, W1.T]])
    b1p = jnp.concatenate([b1, b1]).reshape(1, 2 * _EMBED)
    zc = jnp.zeros((_EMBED, 1), jnp.float32)
    w2p = jnp.block([[W2.T, zc], [zc, W2.T]])
    b2p = jnp.broadcast_to(b2.reshape(1, 1), (1, 2))
    out2 = _tc_head(x, sp, w1p, b1p, w2p, b2p)
    return jnp.concatenate([out2[:, :1], out2[:, 1:]], axis=0)


# speedup vs baseline: 131.2238x; 131.2238x over previous
import jax
import jax.numpy as jnp
from jax import lax
from jax.experimental import pallas as pl
from jax.experimental.pallas import tpu as pltpu
from jax.experimental.pallas import tpu_sc as plsc

_VOCAB = 1000000
_EMBED = 64
_BATCH = 16384
_SEQ = 200
_ROWS = 128
_NUM_BLOCKS = _BATCH // _ROWS
_NC, _NS = 2, 16
_NW = _NC * _NS
_BPW = _NUM_BLOCKS // _NW
_HALF = _BATCH // 2
_HBLK = _NUM_BLOCKS // 2


def _sc_body(x_hbm, table_hbm, out_hbm, xrow_v, idx_v, acc_v, sem_idx,
             sem_g):
    wid = lax.axis_index("s") * _NC + lax.axis_index("c")
    lanes = jax.lax.iota(jnp.int32, 16)

    def _build_idx(slot):
        def _bld(s, carry):
            cols = jnp.full((16,), 0, jnp.int32) + s
            for i0 in range(8):
                v = plsc.load_gather(
                    xrow_v.at[slot], [lanes + (16 * i0), cols])
                idx_v[slot, s, pl.ds(16 * i0, 16)] = v
            return carry
        lax.fori_loop(0, _SEQ, _bld, 0)

    pltpu.sync_copy(
        x_hbm.at[pl.ds(wid * _BPW * _ROWS, _ROWS)], xrow_v.at[0])
    _build_idx(0)

    for t in range(_BPW):
        slot = t % 2
        g = wid * _BPW + t

        pltpu.async_copy(
            table_hbm.at[idx_v.at[slot, 0]], acc_v, sem_g).wait()

        def _fire(s, carry):
            pltpu.async_copy(
                table_hbm.at[idx_v.at[slot, s]], acc_v, sem_g, add=True)
            return carry
        lax.fori_loop(1, _SEQ, _fire, 0)

        if t + 1 < _BPW:
            pltpu.async_copy(
                x_hbm.at[pl.ds((g + 1) * _ROWS, _ROWS)],
                xrow_v.at[1 - slot], sem_idx).wait()
            _build_idx(1 - slot)

        def _drain(s, carry):
            pltpu.make_async_copy(
                table_hbm.at[idx_v.at[slot, 0]], acc_v, sem_g).wait()
            return carry
        lax.fori_loop(1, _SEQ, _drain, 0)

        pltpu.sync_copy(
            acc_v,
            out_hbm.at[pl.ds((g % _HBLK) * _ROWS, _ROWS),
                       pl.ds(_EMBED * (g // _HBLK), _EMBED)])


def _sc_pool(x, table):
    mesh = plsc.VectorSubcoreMesh(core_axis_name="c", subcore_axis_name="s")
    f = pl.kernel(
        _sc_body,
        out_type=jax.ShapeDtypeStruct((_HALF, 2 * _EMBED), jnp.float32),
        mesh=mesh,
        scratch_types=[
            pltpu.VMEM((2, _ROWS, _SEQ), jnp.int32),
            pltpu.VMEM((2, _SEQ, _ROWS), jnp.int32),
            pltpu.VMEM((_ROWS, _EMBED), jnp.float32),
            pltpu.SemaphoreType.DMA,
            pltpu.SemaphoreType.DMA,
        ],
        compiler_params=pltpu.CompilerParams(
            use_tc_tiling_on_sc=False, needs_layout_passes=False),
    )
    return f(x, table)


def _tc_head_body(xa_ref, xb_ref, sp_ref, w1p_ref, b1p_ref, w2p_ref, b2_ref,
                  o_ref):
    cnt_a = jnp.sum((xa_ref[...] != 0).astype(jnp.float32), axis=1,
                    keepdims=True)
    cnt_b = jnp.sum((xb_ref[...] != 0).astype(jnp.float32), axis=1,
                    keepdims=True)
    n = sp_ref.shape[0]
    inv = jnp.concatenate(
        [jnp.broadcast_to(1.0 / jnp.maximum(cnt_a, 1.0), (n, _EMBED)),
         jnp.broadcast_to(1.0 / jnp.maximum(cnt_b, 1.0), (n, _EMBED))],
        axis=1)
    pooled = sp_ref[...] * inv
    h = jnp.dot(pooled, w1p_ref[...], preferred_element_type=jnp.float32)
    h = jnp.maximum(h + b1p_ref[...], 0.0)
    o_ref[...] = (
        jnp.dot(h, w2p_ref[...], preferred_element_type=jnp.float32)
        + b2_ref[...])


def _tc_head(x, sp, w1p, b1p, w2p, b2):
    blk = 1024
    nblk = _HALF // blk
    return pl.pallas_call(
        _tc_head_body,
        grid=(nblk,),
        in_specs=[
            pl.BlockSpec((blk, _SEQ), lambda i: (i, 0)),
            pl.BlockSpec((blk, _SEQ), lambda i: (i + nblk, 0)),
            pl.BlockSpec((blk, 2 * _EMBED), lambda i: (i, 0)),
            pl.BlockSpec((2 * _EMBED, 2 * _EMBED), lambda i: (0, 0)),
            pl.BlockSpec((1, 2 * _EMBED), lambda i: (0, 0)),
            pl.BlockSpec((2 * _EMBED, 2), lambda i: (0, 0)),
            pl.BlockSpec((1, 2), lambda i: (0, 0)),
        ],
        out_specs=pl.BlockSpec((blk, 2), lambda i: (i, 0)),
        out_shape=jax.ShapeDtypeStruct((_HALF, 2), jnp.float32),
    )(x, x, sp, w1p, b1p, w2p, b2)


def kernel(x, table, W1, b1, W2, b2):
    sp = _sc_pool(x, table)
    z = jnp.zeros((_EMBED, _EMBED), jnp.float32)
    w1p = jnp.block([[W1.T, z], [z, W1.T]])
    b1p = jnp.concatenate([b1, b1]).reshape(1, 2 * _EMBED)
    zc = jnp.zeros((_EMBED, 1), jnp.float32)
    w2p = jnp.block([[W2.T, zc], [zc, W2.T]])
    b2p = jnp.broadcast_to(b2.reshape(1, 1), (1, 2))
    out2 = _tc_head(x, sp, w1p, b1p, w2p, b2p)
    return jnp.concatenate([out2[:, :1], out2[:, 1:]], axis=0)


# speedup vs baseline: 136.1283x; 1.0374x over previous
import jax
import jax.numpy as jnp
from jax import lax
from jax.experimental import pallas as pl
from jax.experimental.pallas import tpu as pltpu
from jax.experimental.pallas import tpu_sc as plsc

_VOCAB = 1000000
_EMBED = 64
_BATCH = 16384
_SEQ = 200
_ROWS = 128
_NUM_BLOCKS = _BATCH // _ROWS
_NC, _NS = 2, 16
_NW = _NC * _NS
_BPW = _NUM_BLOCKS // _NW
_HALF = _BATCH // 2
_HBLK = _NUM_BLOCKS // 2


_BLK_IDX = _SEQ * _ROWS


def _sc_body(xb_hbm, table_hbm, out_hbm, idx_v, acc_v, sem_idx, sem_g):
    wid = lax.axis_index("s") * _NC + lax.axis_index("c")

    pltpu.sync_copy(
        xb_hbm.at[pl.ds(wid * _BPW * _BLK_IDX, _BLK_IDX)], idx_v.at[0])

    for t in range(_BPW):
        slot = t % 2
        g = wid * _BPW + t
        if t + 1 < _BPW:
            idx_cp = pltpu.async_copy(
                xb_hbm.at[pl.ds((g + 1) * _BLK_IDX, _BLK_IDX)],
                idx_v.at[1 - slot], sem_idx)

        pltpu.async_copy(
            table_hbm.at[idx_v.at[slot, pl.ds(0, _ROWS)]], acc_v,
            sem_g).wait()

        def _fire(s, carry):
            pltpu.async_copy(
                table_hbm.at[idx_v.at[slot, pl.ds(s * _ROWS, _ROWS)]],
                acc_v, sem_g, add=True)
            return carry
        lax.fori_loop(1, _SEQ, _fire, 0)

        def _drain(s, carry):
            pltpu.make_async_copy(
                table_hbm.at[idx_v.at[slot, pl.ds(0, _ROWS)]], acc_v,
                sem_g).wait()
            return carry
        lax.fori_loop(1, _SEQ, _drain, 0)

        pltpu.sync_copy(
            acc_v,
            out_hbm.at[pl.ds((g % _HBLK) * _ROWS, _ROWS),
                       pl.ds(_EMBED * (g // _HBLK), _EMBED)])
        if t + 1 < _BPW:
            idx_cp.wait()


def _sc_pool(xb, table):
    mesh = plsc.VectorSubcoreMesh(core_axis_name="c", subcore_axis_name="s")
    f = pl.kernel(
        _sc_body,
        out_type=jax.ShapeDtypeStruct((_HALF, 2 * _EMBED), jnp.float32),
        mesh=mesh,
        scratch_types=[
            pltpu.VMEM((2, _BLK_IDX), jnp.int32),
            pltpu.VMEM((_ROWS, _EMBED), jnp.float32),
            pltpu.SemaphoreType.DMA,
            pltpu.SemaphoreType.DMA,
        ],
        compiler_params=pltpu.CompilerParams(use_tc_tiling_on_sc=False),
    )
    return f(xb, table)


def _tc_transpose_body(x_ref, o_ref):
    o_ref[0] = x_ref[...].T


def _tc_transpose(x):
    return pl.pallas_call(
        _tc_transpose_body,
        grid=(_NUM_BLOCKS,),
        in_specs=[pl.BlockSpec((_ROWS, _SEQ), lambda i: (i, 0))],
        out_specs=pl.BlockSpec((1, _SEQ, _ROWS), lambda i: (i, 0, 0)),
        out_shape=jax.ShapeDtypeStruct((_NUM_BLOCKS, _SEQ, _ROWS), jnp.int32),
    )(x)


def _tc_head_body(xa_ref, xb_ref, sp_ref, w1p_ref, b1p_ref, w2p_ref, b2_ref,
                  o_ref):
    cnt_a = jnp.sum((xa_ref[...] != 0).astype(jnp.float32), axis=1,
                    keepdims=True)
    cnt_b = jnp.sum((xb_ref[...] != 0).astype(jnp.float32), axis=1,
                    keepdims=True)
    n = sp_ref.shape[0]
    inv = jnp.concatenate(
        [jnp.broadcast_to(1.0 / jnp.maximum(cnt_a, 1.0), (n, _EMBED)),
         jnp.broadcast_to(1.0 / jnp.maximum(cnt_b, 1.0), (n, _EMBED))],
        axis=1)
    pooled = sp_ref[...] * inv
    h = jnp.dot(pooled, w1p_ref[...], preferred_element_type=jnp.float32)
    h = jnp.maximum(h + b1p_ref[...], 0.0)
    o_ref[...] = (
        jnp.dot(h, w2p_ref[...], preferred_element_type=jnp.float32)
        + b2_ref[...])


def _tc_head(x, sp, w1p, b1p, w2p, b2):
    blk = 1024
    nblk = _HALF // blk
    return pl.pallas_call(
        _tc_head_body,
        grid=(nblk,),
        in_specs=[
            pl.BlockSpec((blk, _SEQ), lambda i: (i, 0)),
            pl.BlockSpec((blk, _SEQ), lambda i: (i + nblk, 0)),
            pl.BlockSpec((blk, 2 * _EMBED), lambda i: (i, 0)),
            pl.BlockSpec((2 * _EMBED, 2 * _EMBED), lambda i: (0, 0)),
            pl.BlockSpec((1, 2 * _EMBED), lambda i: (0, 0)),
            pl.BlockSpec((2 * _EMBED, 2), lambda i: (0, 0)),
            pl.BlockSpec((1, 2), lambda i: (0, 0)),
        ],
        out_specs=pl.BlockSpec((blk, 2), lambda i: (i, 0)),
        out_shape=jax.ShapeDtypeStruct((_HALF, 2), jnp.float32),
    )(x, x, sp, w1p, b1p, w2p, b2)


def kernel(x, table, W1, b1, W2, b2):
    xb = _tc_transpose(x).reshape(-1)
    sp = _sc_pool(xb, table)
    z = jnp.zeros((_EMBED, _EMBED), jnp.float32)
    w1p = jnp.block([[W1.T, z], [z, W1.T]])
    b1p = jnp.concatenate([b1, b1]).reshape(1, 2 * _EMBED)
    zc = jnp.zeros((_EMBED, 1), jnp.float32)
    w2p = jnp.block([[W2.T, zc], [zc, W2.T]])
    b2p = jnp.broadcast_to(b2.reshape(1, 1), (1, 2))
    out2 = _tc_head(x, sp, w1p, b1p, w2p, b2p)
    return jnp.concatenate([out2[:, :1], out2[:, 1:]], axis=0)
